# Initial kernel scaffold; baseline (speedup 1.0000x reference)
import jax
import jax.numpy as jnp
from jax.experimental import pallas as pl


def kernel(x, edge_index, W1, a_src1, a_dst1, b1, W2, a_src2, a_dst2, b2, W3, a_src3, a_dst3, b3):
    raise NotImplementedError("write your pallas kernel here")


# speedup vs baseline: 26.7710x; 26.7710x over previous
import functools

import jax
import jax.numpy as jnp
from jax import lax
from jax.experimental import pallas as pl
from jax.experimental.pallas import tpu as pltpu
from jax.experimental.pallas import tpu_sc as plsc

N = 10000
NP = 10240
TR = NP // 16
K = 128
NSUB = 16
E_REAL = 650000
EP = ((E_REAL + NSUB * K - 1) // (NSUB * K)) * (NSUB * K)
EPT = EP // NSUB
NB = EPT // K
BR = 2048


def _lrelu(x, s):
  return jnp.where(x >= 0, x, s * x)


def _tc_first_body(x_ref, w_ref, as_ref, ad_ref,
                   h_ref, als_ref, ald_ref, mx_ref):
  i = pl.program_id(0)
  h = jnp.dot(x_ref[...], w_ref[...], preferred_element_type=jnp.float32)
  h_ref[...] = h
  als = jnp.dot(h, as_ref[...], preferred_element_type=jnp.float32)
  ald = jnp.dot(h, ad_ref[...], preferred_element_type=jnp.float32)
  als_ref[...] = als
  ald_ref[...] = ald

  @pl.when(i == 0)
  def _():
    mx_ref[...] = jnp.full((1, 1), -1e30, jnp.float32)

  mx_ref[...] = jnp.maximum(mx_ref[...], jnp.max(als))


def _tc_mid_body(acc_ref, den_ref, b_ref, w_ref, as_ref, ad_ref,
                 h_ref, als_ref, ald_ref, mx_ref):
  i = pl.program_id(0)
  g = _lrelu(acc_ref[...] / (den_ref[...] + 1e-16) + b_ref[...], 0.01)
  row = i * BR + lax.broadcasted_iota(jnp.int32, (BR, 1), 0)
  g = jnp.where(row < N, g, 0.0)
  h = jnp.dot(g, w_ref[...], preferred_element_type=jnp.float32)
  h_ref[...] = h
  als = jnp.dot(h, as_ref[...], preferred_element_type=jnp.float32)
  ald = jnp.dot(h, ad_ref[...], preferred_element_type=jnp.float32)
  als_ref[...] = als
  ald_ref[...] = ald

  @pl.when(i == 0)
  def _():
    mx_ref[...] = jnp.full((1, 1), -1e30, jnp.float32)

  mx_ref[...] = jnp.maximum(mx_ref[...], jnp.max(als))


def _tc_first(x, w, a_s, a_d):
  din, dout = w.shape
  grid = NP // BR
  return pl.pallas_call(
      _tc_first_body,
      grid=(grid,),
      in_specs=[
          pl.BlockSpec((BR, din), lambda i: (i, 0)),
          pl.BlockSpec((din, dout), lambda i: (0, 0)),
          pl.BlockSpec((dout, 1), lambda i: (0, 0)),
          pl.BlockSpec((dout, 1), lambda i: (0, 0)),
      ],
      out_specs=[
          pl.BlockSpec((BR, dout), lambda i: (i, 0)),
          pl.BlockSpec((BR, 1), lambda i: (i, 0)),
          pl.BlockSpec((BR, 1), lambda i: (i, 0)),
          pl.BlockSpec((1, 1), lambda i: (0, 0)),
      ],
      out_shape=[
          jax.ShapeDtypeStruct((NP, dout), jnp.float32),
          jax.ShapeDtypeStruct((NP, 1), jnp.float32),
          jax.ShapeDtypeStruct((NP, 1), jnp.float32),
          jax.ShapeDtypeStruct((1, 1), jnp.float32),
      ],
  )(x, w, a_s, a_d)


def _tc_mid(acc, den, b, w, a_s, a_d):
  din, dout = w.shape
  grid = NP // BR
  return pl.pallas_call(
      _tc_mid_body,
      grid=(grid,),
      in_specs=[
          pl.BlockSpec((BR, din), lambda i: (i, 0)),
          pl.BlockSpec((BR, 1), lambda i: (i, 0)),
          pl.BlockSpec((1, din), lambda i: (0, 0)),
          pl.BlockSpec((din, dout), lambda i: (0, 0)),
          pl.BlockSpec((dout, 1), lambda i: (0, 0)),
          pl.BlockSpec((dout, 1), lambda i: (0, 0)),
      ],
      out_specs=[
          pl.BlockSpec((BR, dout), lambda i: (i, 0)),
          pl.BlockSpec((BR, 1), lambda i: (i, 0)),
          pl.BlockSpec((BR, 1), lambda i: (i, 0)),
          pl.BlockSpec((1, 1), lambda i: (0, 0)),
      ],
      out_shape=[
          jax.ShapeDtypeStruct((NP, dout), jnp.float32),
          jax.ShapeDtypeStruct((NP, 1), jnp.float32),
          jax.ShapeDtypeStruct((NP, 1), jnp.float32),
          jax.ShapeDtypeStruct((1, 1), jnp.float32),
      ],
  )(acc, den, b, w, a_s, a_d)


def _iota16():
  return lax.iota(jnp.int32, 16)


def _edge_weights(base, mxval, als_v, ald_v, sidx_v, didx_v, ex_v):
  for j in range(K // 16):
    si = sidx_v[pl.ds(j * 16, 16)]
    di = didx_v[pl.ds(j * 16, 16)]
    asg = plsc.load_gather(als_v, [si])
    adg = plsc.load_gather(ald_v, [di])
    e = _lrelu(asg + adg, 0.2)
    m = _lrelu(mxval + adg, 0.2)
    ex = jnp.exp(e - m)
    gi = base + (j * 16) + _iota16()
    ex = jnp.where(gi < E_REAL, ex, 0.0)
    ex_v[pl.ds(j * 16, 16)] = ex


def _sc_agg_body(d2, ha_hbm, hb_hbm, als_hbm, ald_hbm, mx_hbm, src_hbm,
                 dst_hbm, acc_hbm, den_hbm,
                 als_v, ald_v, mx_v, sidx_v, didx_v, ex_v, rows_v, zden_v,
                 acc_s, den_s, sem):
  c = lax.axis_index("c")
  s = lax.axis_index("s")

  pltpu.sync_copy(als_hbm, als_v)
  pltpu.sync_copy(ald_hbm, ald_v)
  pltpu.sync_copy(mx_hbm, mx_v)

  def _zrows(i, _):
    for cc in range(d2 // 16):
      rows_v[i, pl.ds(cc * 16, 16)] = jnp.zeros((16,), jnp.float32)
    return 0
  lax.fori_loop(0, K, _zrows, 0)

  def _zden(i, _):
    zden_v[pl.ds(i * 16, 16)] = jnp.zeros((16,), jnp.float32)
    return 0
  lax.fori_loop(0, TR // 16, _zden, 0)

  for j in range(TR // K):
    pltpu.sync_copy(rows_v, acc_s.at[pl.ds(s * TR + j * K, K)])
  pltpu.sync_copy(zden_v, den_s.at[pl.ds(s * TR, TR)])
  plsc.subcore_barrier()

  mxval = mx_v[pl.ds(0, 16)]

  def _batch(b, _):
    base = s * EPT + b * K
    pltpu.sync_copy(src_hbm.at[pl.ds(base, K)], sidx_v)
    pltpu.sync_copy(dst_hbm.at[pl.ds(base, K)], didx_v)

    @pl.when(c == 0)
    def _():
      pltpu.async_copy(ha_hbm.at[sidx_v], rows_v, sem)

    @pl.when(c == 1)
    def _():
      pltpu.async_copy(hb_hbm.at[sidx_v], rows_v, sem)

    _edge_weights(base, mxval, als_v, ald_v, sidx_v, didx_v, ex_v)

    pltpu.make_async_copy(ha_hbm.at[sidx_v], rows_v, sem).wait()

    def _scale(k, _):
      exk = plsc.load_gather(ex_v, [jnp.full((16,), 0, jnp.int32) + k])
      for cc in range(d2 // 16):
        rows_v[k, pl.ds(cc * 16, 16)] = rows_v[k, pl.ds(cc * 16, 16)] * exk
      return 0
    lax.fori_loop(0, K, _scale, 0)

    pltpu.sync_copy(rows_v, acc_s.at[didx_v], add=True)

    @pl.when(c == 0)
    def _():
      pltpu.sync_copy(ex_v, den_s.at[didx_v], add=True)

    return 0

  lax.fori_loop(0, NB, _batch, 0)
  plsc.subcore_barrier()

  pltpu.sync_copy(acc_s.at[pl.ds(s * TR, TR)],
                  acc_hbm.at[c, pl.ds(s * TR, TR)])

  @pl.when(c == 0)
  def _():
    pltpu.sync_copy(den_s.at[pl.ds(s * TR, TR)], den_hbm.at[pl.ds(s * TR, TR)])


def _sc_agg(ha, hb, als, ald, mxv, src, dst):
  d2 = ha.shape[1]
  mesh = plsc.VectorSubcoreMesh(core_axis_name="c", subcore_axis_name="s",
                                num_cores=2, num_subcores=NSUB)
  fn = pl.kernel(
      functools.partial(_sc_agg_body, d2),
      out_type=[
          jax.ShapeDtypeStruct((2, NP, d2), jnp.float32),
          jax.ShapeDtypeStruct((NP,), jnp.float32),
      ],
      mesh=mesh,
      compiler_params=pltpu.CompilerParams(needs_layout_passes=False),
      scratch_types=[
          pltpu.VMEM((NP,), jnp.float32),
          pltpu.VMEM((NP,), jnp.float32),
          pltpu.VMEM((16,), jnp.float32),
          pltpu.VMEM((K,), jnp.int32),
          pltpu.VMEM((K,), jnp.int32),
          pltpu.VMEM((K,), jnp.float32),
          pltpu.VMEM((K, d2), jnp.float32),
          pltpu.VMEM((TR,), jnp.float32),
          pltpu.VMEM_SHARED((NP, d2), jnp.float32),
          pltpu.VMEM_SHARED((NP,), jnp.float32),
          pltpu.SemaphoreType.DMA,
      ],
  )
  return fn(ha, hb, als, ald, mxv, src, dst)


def _sc_agg2_body(h_hbm, als_hbm, ald_hbm, mx_hbm, src_hbm,
                  dst_hbm, acc_hbm, den_hbm,
                  als_v, ald_v, mx_v, sidx_v, didx_v, ex_v, rows_v, zden_v,
                  acc_s, den_s, sem):
  d = rows_v.shape[1]
  c = lax.axis_index("c")
  s = lax.axis_index("s")

  pltpu.sync_copy(als_hbm, als_v)
  pltpu.sync_copy(ald_hbm, ald_v)
  pltpu.sync_copy(mx_hbm, mx_v)

  def _zrows(i, _):
    for cc in range(d // 16):
      rows_v[i, pl.ds(cc * 16, 16)] = jnp.zeros((16,), jnp.float32)
    return 0
  lax.fori_loop(0, K, _zrows, 0)

  def _zden(i, _):
    zden_v[pl.ds(i * 16, 16)] = jnp.zeros((16,), jnp.float32)
    return 0
  lax.fori_loop(0, TR // 16, _zden, 0)

  for j in range(TR // K):
    pltpu.sync_copy(rows_v, acc_s.at[pl.ds(s * TR + j * K, K)])
  pltpu.sync_copy(zden_v, den_s.at[pl.ds(s * TR, TR)])
  plsc.subcore_barrier()

  mxval = mx_v[pl.ds(0, 16)]
  ept2 = EP // 32
  nb2 = ept2 // K

  def _batch(b, _):
    base = (s * 2 + c) * ept2 + b * K
    pltpu.sync_copy(src_hbm.at[pl.ds(base, K)], sidx_v)
    pltpu.sync_copy(dst_hbm.at[pl.ds(base, K)], didx_v)
    pltpu.async_copy(h_hbm.at[sidx_v], rows_v, sem)
    _edge_weights(base, mxval, als_v, ald_v, sidx_v, didx_v, ex_v)
    pltpu.make_async_copy(h_hbm.at[sidx_v], rows_v, sem).wait()

    def _scale(k, _):
      exk = plsc.load_gather(ex_v, [jnp.full((16,), 0, jnp.int32) + k])
      for cc in range(d // 16):
        rows_v[k, pl.ds(cc * 16, 16)] = rows_v[k, pl.ds(cc * 16, 16)] * exk
      return 0
    lax.fori_loop(0, K, _scale, 0)

    pltpu.sync_copy(rows_v, acc_s.at[didx_v], add=True)
    pltpu.sync_copy(ex_v, den_s.at[didx_v], add=True)
    return 0

  lax.fori_loop(0, nb2, _batch, 0)
  plsc.subcore_barrier()

  pltpu.sync_copy(acc_s.at[pl.ds(s * TR, TR)],
                  acc_hbm.at[c, pl.ds(s * TR, TR)])
  pltpu.sync_copy(den_s.at[pl.ds(s * TR, TR)],
                  den_hbm.at[c, pl.ds(s * TR, TR)])


def _sc_agg2(h, als, ald, mxv, src, dst):
  d = h.shape[1]
  mesh = plsc.VectorSubcoreMesh(core_axis_name="c", subcore_axis_name="s",
                                num_cores=2, num_subcores=NSUB)
  fn = pl.kernel(
      _sc_agg2_body,
      out_type=[
          jax.ShapeDtypeStruct((2, NP, d), jnp.float32),
          jax.ShapeDtypeStruct((2, NP), jnp.float32),
      ],
      mesh=mesh,
      compiler_params=pltpu.CompilerParams(needs_layout_passes=False),
      scratch_types=[
          pltpu.VMEM((NP,), jnp.float32),
          pltpu.VMEM((NP,), jnp.float32),
          pltpu.VMEM((16,), jnp.float32),
          pltpu.VMEM((K,), jnp.int32),
          pltpu.VMEM((K,), jnp.int32),
          pltpu.VMEM((K,), jnp.float32),
          pltpu.VMEM((K, d), jnp.float32),
          pltpu.VMEM((TR,), jnp.float32),
          pltpu.VMEM_SHARED((NP, d), jnp.float32),
          pltpu.VMEM_SHARED((NP,), jnp.float32),
          pltpu.SemaphoreType.DMA,
      ],
  )
  return fn(h, als, ald, mxv, src, dst)


def _tc_mid2_body(a0_ref, a1_ref, d0_ref, d1_ref, b_ref, w_ref, as_ref, ad_ref,
                  h_ref, als_ref, ald_ref, mx_ref):
  i = pl.program_id(0)
  acc = a0_ref[...] + a1_ref[...]
  den = d0_ref[...] + d1_ref[...]
  g = _lrelu(acc / (den + 1e-16) + b_ref[...], 0.01)
  row = i * BR + lax.broadcasted_iota(jnp.int32, (BR, 1), 0)
  g = jnp.where(row < N, g, 0.0)
  h = jnp.dot(g, w_ref[...], preferred_element_type=jnp.float32)
  h_ref[...] = h
  als = jnp.dot(h, as_ref[...], preferred_element_type=jnp.float32)
  ald = jnp.dot(h, ad_ref[...], preferred_element_type=jnp.float32)
  als_ref[...] = als
  ald_ref[...] = ald

  @pl.when(i == 0)
  def _():
    mx_ref[...] = jnp.full((1, 1), -1e30, jnp.float32)

  mx_ref[...] = jnp.maximum(mx_ref[...], jnp.max(als))


def _tc_mid2(a0, a1, d0, d1, b, w, a_s, a_d):
  din, dout = w.shape
  grid = NP // BR
  return pl.pallas_call(
      _tc_mid2_body,
      grid=(grid,),
      in_specs=[
          pl.BlockSpec((BR, din), lambda i: (i, 0)),
          pl.BlockSpec((BR, din), lambda i: (i, 0)),
          pl.BlockSpec((BR, 1), lambda i: (i, 0)),
          pl.BlockSpec((BR, 1), lambda i: (i, 0)),
          pl.BlockSpec((1, din), lambda i: (0, 0)),
          pl.BlockSpec((din, dout), lambda i: (0, 0)),
          pl.BlockSpec((dout, 1), lambda i: (0, 0)),
          pl.BlockSpec((dout, 1), lambda i: (0, 0)),
      ],
      out_specs=[
          pl.BlockSpec((BR, dout), lambda i: (i, 0)),
          pl.BlockSpec((BR, 1), lambda i: (i, 0)),
          pl.BlockSpec((BR, 1), lambda i: (i, 0)),
          pl.BlockSpec((1, 1), lambda i: (0, 0)),
      ],
      out_shape=[
          jax.ShapeDtypeStruct((NP, dout), jnp.float32),
          jax.ShapeDtypeStruct((NP, 1), jnp.float32),
          jax.ShapeDtypeStruct((NP, 1), jnp.float32),
          jax.ShapeDtypeStruct((1, 1), jnp.float32),
      ],
  )(a0, a1, d0, d1, b, w, a_s, a_d)


def _sc_final_body(h3_hbm, als_hbm, ald_hbm, mx_hbm, b3_hbm, src_hbm, dst_hbm,
                   out_hbm,
                   h3_v, als_v, ald_v, mx_v, b3_v, sidx_v, didx_v, ex_v, val_v,
                   numv_v, denv_v, outv_v, num_s, den_s):
  c = lax.axis_index("c")
  s = lax.axis_index("s")

  @pl.when(c == 0)
  def _():
    pltpu.sync_copy(h3_hbm, h3_v)
    pltpu.sync_copy(als_hbm, als_v)
    pltpu.sync_copy(ald_hbm, ald_v)
    pltpu.sync_copy(mx_hbm, mx_v)
    pltpu.sync_copy(b3_hbm, b3_v)

    def _zd(i, _):
      numv_v[pl.ds(i * 16, 16)] = jnp.zeros((16,), jnp.float32)
      denv_v[pl.ds(i * 16, 16)] = jnp.zeros((16,), jnp.float32)
      return 0
    lax.fori_loop(0, TR // 16, _zd, 0)
    pltpu.sync_copy(numv_v, num_s.at[pl.ds(s * TR, TR)])
    pltpu.sync_copy(denv_v, den_s.at[pl.ds(s * TR, TR)])
    plsc.subcore_barrier()

    mxval = mx_v[pl.ds(0, 16)]

    def _batch(b, _):
      base = s * EPT + b * K
      pltpu.sync_copy(src_hbm.at[pl.ds(base, K)], sidx_v)
      pltpu.sync_copy(dst_hbm.at[pl.ds(base, K)], didx_v)
      for j in range(K // 16):
        si = sidx_v[pl.ds(j * 16, 16)]
        di = didx_v[pl.ds(j * 16, 16)]
        asg = plsc.load_gather(als_v, [si])
        adg = plsc.load_gather(ald_v, [di])
        e = _lrelu(asg + adg, 0.2)
        m = _lrelu(mxval + adg, 0.2)
        ex = jnp.exp(e - m)
        gi = base + (j * 16) + _iota16()
        ex = jnp.where(gi < E_REAL, ex, 0.0)
        h3g = plsc.load_gather(h3_v, [si])
        ex_v[pl.ds(j * 16, 16)] = ex
        val_v[pl.ds(j * 16, 16)] = ex * h3g
      pltpu.sync_copy(val_v, num_s.at[didx_v], add=True)
      pltpu.sync_copy(ex_v, den_s.at[didx_v], add=True)
      return 0

    lax.fori_loop(0, NB, _batch, 0)
    plsc.subcore_barrier()

    pltpu.sync_copy(num_s.at[pl.ds(s * TR, TR)], numv_v)
    pltpu.sync_copy(den_s.at[pl.ds(s * TR, TR)], denv_v)
    b3val = b3_v[pl.ds(0, 16)]

    def _fin(i, _):
      num = numv_v[pl.ds(i * 16, 16)]
      den = denv_v[pl.ds(i * 16, 16)]
      outv_v[pl.ds(i * 16, 16)] = num / (den + 1e-16) + b3val
      return 0
    lax.fori_loop(0, TR // 16, _fin, 0)
    pltpu.sync_copy(outv_v, out_hbm.at[pl.ds(s * TR, TR)])


def _sc_final(h3, als, ald, mxv, b3v, src, dst):
  mesh = plsc.VectorSubcoreMesh(core_axis_name="c", subcore_axis_name="s",
                                num_cores=2, num_subcores=NSUB)
  fn = pl.kernel(
      _sc_final_body,
      out_type=jax.ShapeDtypeStruct((NP,), jnp.float32),
      mesh=mesh,
      compiler_params=pltpu.CompilerParams(needs_layout_passes=False),
      scratch_types=[
          pltpu.VMEM((NP,), jnp.float32),
          pltpu.VMEM((NP,), jnp.float32),
          pltpu.VMEM((NP,), jnp.float32),
          pltpu.VMEM((16,), jnp.float32),
          pltpu.VMEM((16,), jnp.float32),
          pltpu.VMEM((K,), jnp.int32),
          pltpu.VMEM((K,), jnp.int32),
          pltpu.VMEM((K,), jnp.float32),
          pltpu.VMEM((K,), jnp.float32),
          pltpu.VMEM((TR,), jnp.float32),
          pltpu.VMEM((TR,), jnp.float32),
          pltpu.VMEM((TR,), jnp.float32),
          pltpu.VMEM_SHARED((NP,), jnp.float32),
          pltpu.VMEM_SHARED((NP,), jnp.float32),
      ],
  )
  return fn(h3, als, ald, mxv, b3v, src, dst)


def kernel(x, edge_index, W1, a_src1, a_dst1, b1,
           W2, a_src2, a_dst2, b2, W3, a_src3, a_dst3, b3):
  f32 = jnp.float32
  xp = jnp.pad(x.astype(f32), ((0, NP - N), (0, 0)))
  loop = jnp.arange(N, dtype=jnp.int32)
  padi = jnp.zeros((EP - E_REAL,), jnp.int32)
  src = jnp.concatenate([edge_index[0].astype(jnp.int32), loop, padi])
  dst = jnp.concatenate([edge_index[1].astype(jnp.int32), loop, padi])

  h1, als1, ald1, mx1 = _tc_first(xp, W1, a_src1.reshape(-1, 1),
                                  a_dst1.reshape(-1, 1))
  mx1v = jnp.broadcast_to(mx1.reshape(1), (16,))
  d2 = h1.shape[1] // 2
  acc1, den1 = _sc_agg(h1[:, :d2], h1[:, d2:], als1[:, 0], ald1[:, 0],
                       mx1v, src, dst)
  accf1 = jnp.concatenate([acc1[0], acc1[1]], axis=1)

  h2, als2, ald2, mx2 = _tc_mid(accf1, den1.reshape(-1, 1), b1.reshape(1, -1),
                                W2, a_src2.reshape(-1, 1), a_dst2.reshape(-1, 1))
  mx2v = jnp.broadcast_to(mx2.reshape(1), (16,))
  acc2, den2 = _sc_agg2(h2, als2[:, 0], ald2[:, 0], mx2v, src, dst)

  h3, als3, ald3, mx3 = _tc_mid2(acc2[0], acc2[1],
                                 den2[0].reshape(-1, 1), den2[1].reshape(-1, 1),
                                 b2.reshape(1, -1),
                                 W3, a_src3.reshape(-1, 1), a_dst3.reshape(-1, 1))
  mx3v = jnp.broadcast_to(mx3.reshape(1), (16,))
  b3v = jnp.broadcast_to(b3.reshape(1), (16,))
  outp = _sc_final(h3[:, 0], als3[:, 0], ald3[:, 0], mx3v, b3v, src, dst)
  return outp[:N]


# speedup vs baseline: 30.6166x; 1.1436x over previous
import functools

import jax
import jax.numpy as jnp
from jax import lax
from jax.experimental import pallas as pl
from jax.experimental.pallas import tpu as pltpu
from jax.experimental.pallas import tpu_sc as plsc

N = 10000
NP = 10240
TR = NP // 16
K = 128
NSUB = 16
E_REAL = 650000
EP = 655360
EPT = EP // NSUB
NB = EPT // K
BR = 2048


def _lrelu(x, s):
  return jnp.where(x >= 0, x, s * x)


def _tc_first_body(x_ref, w_ref, as_ref, ad_ref,
                   h_ref, als_ref, ald_ref, mx_ref):
  i = pl.program_id(0)
  h = jnp.dot(x_ref[...], w_ref[...], preferred_element_type=jnp.float32)
  h_ref[...] = h
  als = jnp.dot(h, as_ref[...], preferred_element_type=jnp.float32)
  ald = jnp.dot(h, ad_ref[...], preferred_element_type=jnp.float32)
  als_ref[...] = als
  ald_ref[...] = ald

  @pl.when(i == 0)
  def _():
    mx_ref[...] = jnp.full((1, 1), -1e30, jnp.float32)

  mx_ref[...] = jnp.maximum(mx_ref[...], jnp.max(als))


def _tc_first(x, w, a_s, a_d):
  din, dout = w.shape
  grid = NP // BR
  return pl.pallas_call(
      _tc_first_body,
      grid=(grid,),
      in_specs=[
          pl.BlockSpec((BR, din), lambda i: (i, 0)),
          pl.BlockSpec((din, dout), lambda i: (0, 0)),
          pl.BlockSpec((dout, 1), lambda i: (0, 0)),
          pl.BlockSpec((dout, 1), lambda i: (0, 0)),
      ],
      out_specs=[
          pl.BlockSpec((BR, dout), lambda i: (i, 0)),
          pl.BlockSpec((BR, 1), lambda i: (i, 0)),
          pl.BlockSpec((BR, 1), lambda i: (i, 0)),
          pl.BlockSpec((1, 1), lambda i: (0, 0)),
      ],
      out_shape=[
          jax.ShapeDtypeStruct((NP, dout), jnp.float32),
          jax.ShapeDtypeStruct((NP, 1), jnp.float32),
          jax.ShapeDtypeStruct((NP, 1), jnp.float32),
          jax.ShapeDtypeStruct((1, 1), jnp.float32),
      ],
  )(x, w, a_s, a_d)


def _tc_mid_body(acc_ref, den_ref, b_ref, w_ref, as_ref, ad_ref,
                 h_ref, als_ref, ald_ref, mx_ref):
  i = pl.program_id(0)
  g = _lrelu(acc_ref[...] / (den_ref[...] + 1e-16) + b_ref[...], 0.01)
  row = i * BR + lax.broadcasted_iota(jnp.int32, (BR, 1), 0)
  g = jnp.where(row < N, g, 0.0)
  h = jnp.dot(g, w_ref[...], preferred_element_type=jnp.float32)
  h_ref[...] = h
  als = jnp.dot(h, as_ref[...], preferred_element_type=jnp.float32)
  ald = jnp.dot(h, ad_ref[...], preferred_element_type=jnp.float32)
  als_ref[...] = als
  ald_ref[...] = ald

  @pl.when(i == 0)
  def _():
    mx_ref[...] = jnp.full((1, 1), -1e30, jnp.float32)

  mx_ref[...] = jnp.maximum(mx_ref[...], jnp.max(als))


def _tc_mid(acc, den, b, w, a_s, a_d):
  din, dout = w.shape
  grid = NP // BR
  return pl.pallas_call(
      _tc_mid_body,
      grid=(grid,),
      in_specs=[
          pl.BlockSpec((BR, din), lambda i: (i, 0)),
          pl.BlockSpec((BR, 1), lambda i: (i, 0)),
          pl.BlockSpec((1, din), lambda i: (0, 0)),
          pl.BlockSpec((din, dout), lambda i: (0, 0)),
          pl.BlockSpec((dout, 1), lambda i: (0, 0)),
          pl.BlockSpec((dout, 1), lambda i: (0, 0)),
      ],
      out_specs=[
          pl.BlockSpec((BR, dout), lambda i: (i, 0)),
          pl.BlockSpec((BR, 1), lambda i: (i, 0)),
          pl.BlockSpec((BR, 1), lambda i: (i, 0)),
          pl.BlockSpec((1, 1), lambda i: (0, 0)),
      ],
      out_shape=[
          jax.ShapeDtypeStruct((NP, dout), jnp.float32),
          jax.ShapeDtypeStruct((NP, 1), jnp.float32),
          jax.ShapeDtypeStruct((NP, 1), jnp.float32),
          jax.ShapeDtypeStruct((1, 1), jnp.float32),
      ],
  )(acc, den, b, w, a_s, a_d)


def _tc_mid2_body(a0_ref, a1_ref, d0_ref, d1_ref, b_ref, w_ref, as_ref, ad_ref,
                  h_ref, als_ref, ald_ref, mx_ref):
  i = pl.program_id(0)
  acc = a0_ref[...] + a1_ref[...]
  den = d0_ref[...] + d1_ref[...]
  g = _lrelu(acc / (den + 1e-16) + b_ref[...], 0.01)
  row = i * BR + lax.broadcasted_iota(jnp.int32, (BR, 1), 0)
  g = jnp.where(row < N, g, 0.0)
  h = jnp.dot(g, w_ref[...], preferred_element_type=jnp.float32)
  h_ref[...] = h
  als = jnp.dot(h, as_ref[...], preferred_element_type=jnp.float32)
  ald = jnp.dot(h, ad_ref[...], preferred_element_type=jnp.float32)
  als_ref[...] = als
  ald_ref[...] = ald

  @pl.when(i == 0)
  def _():
    mx_ref[...] = jnp.full((1, 1), -1e30, jnp.float32)

  mx_ref[...] = jnp.maximum(mx_ref[...], jnp.max(als))


def _tc_mid2(a0, a1, d0, d1, b, w, a_s, a_d):
  din, dout = w.shape
  grid = NP // BR
  return pl.pallas_call(
      _tc_mid2_body,
      grid=(grid,),
      in_specs=[
          pl.BlockSpec((BR, din), lambda i: (i, 0)),
          pl.BlockSpec((BR, din), lambda i: (i, 0)),
          pl.BlockSpec((BR, 1), lambda i: (i, 0)),
          pl.BlockSpec((BR, 1), lambda i: (i, 0)),
          pl.BlockSpec((1, din), lambda i: (0, 0)),
          pl.BlockSpec((din, dout), lambda i: (0, 0)),
          pl.BlockSpec((dout, 1), lambda i: (0, 0)),
          pl.BlockSpec((dout, 1), lambda i: (0, 0)),
      ],
      out_specs=[
          pl.BlockSpec((BR, dout), lambda i: (i, 0)),
          pl.BlockSpec((BR, 1), lambda i: (i, 0)),
          pl.BlockSpec((BR, 1), lambda i: (i, 0)),
          pl.BlockSpec((1, 1), lambda i: (0, 0)),
      ],
      out_shape=[
          jax.ShapeDtypeStruct((NP, dout), jnp.float32),
          jax.ShapeDtypeStruct((NP, 1), jnp.float32),
          jax.ShapeDtypeStruct((NP, 1), jnp.float32),
          jax.ShapeDtypeStruct((1, 1), jnp.float32),
      ],
  )(a0, a1, d0, d1, b, w, a_s, a_d)


def _tc_fin_body(n0_ref, n1_ref, d0_ref, d1_ref, b_ref, o_ref):
  num = n0_ref[...] + n1_ref[...]
  den = d0_ref[...] + d1_ref[...]
  o_ref[...] = num / (den + 1e-16) + b_ref[...]


def _tc_fin(n0, n1, d0, d1, b3):
  grid = NP // BR
  return pl.pallas_call(
      _tc_fin_body,
      grid=(grid,),
      in_specs=[
          pl.BlockSpec((BR, 1), lambda i: (i, 0)),
          pl.BlockSpec((BR, 1), lambda i: (i, 0)),
          pl.BlockSpec((BR, 1), lambda i: (i, 0)),
          pl.BlockSpec((BR, 1), lambda i: (i, 0)),
          pl.BlockSpec((1, 1), lambda i: (0, 0)),
      ],
      out_specs=pl.BlockSpec((BR, 1), lambda i: (i, 0)),
      out_shape=jax.ShapeDtypeStruct((NP, 1), jnp.float32),
  )(n0, n1, d0, d1, b3)


def _iota16():
  return lax.iota(jnp.int32, 16)


def _edge_weights(base, mxval, als_v, ald_v, sidx_v, didx_v, ex_v):
  for j in range(K // 16):
    si = sidx_v[pl.ds(j * 16, 16)]
    di = didx_v[pl.ds(j * 16, 16)]
    asg = plsc.load_gather(als_v, [si])
    adg = plsc.load_gather(ald_v, [di])
    e = _lrelu(asg + adg, 0.2)
    m = _lrelu(mxval + adg, 0.2)
    ex = jnp.exp(e - m)
    gi = base + (j * 16) + _iota16()
    ex = jnp.where(gi < E_REAL, ex, 0.0)
    ex_v[pl.ds(j * 16, 16)] = ex


def _sc_pipe_body(mode, ha_hbm, hb_hbm, als_hbm, ald_hbm, mx_hbm, src_hbm,
                  dst_hbm, acc_hbm, den_hbm,
                  als_v, ald_v, mx_v, s0_v, s1_v, d0_v, d1_v, e0_v, e1_v,
                  r_v, zden_v, acc_s, den_s,
                  sem_i0, sem_i1, sem_g):
  d = r_v.shape[1]
  c = lax.axis_index("c")
  s = lax.axis_index("s")
  sidx = (s0_v, s1_v)
  didx = (d0_v, d1_v)
  exv = (e0_v, e1_v)
  semi = (sem_i0, sem_i1)

  if mode == "feat":
    ept = EPT
    nb = NB
    ebase = lambda b: s * ept + b * K
  else:
    ept = EP // 32
    nb = ept // K
    ebase = lambda b: (s * 2 + c) * ept + b * K

  def fire_i(b, p):
    base = ebase(b)
    pltpu.async_copy(src_hbm.at[pl.ds(base, K)], sidx[p], semi[p])
    pltpu.async_copy(dst_hbm.at[pl.ds(base, K)], didx[p], semi[p])

  def wait_i(p):
    pltpu.make_async_copy(src_hbm.at[pl.ds(0, K)], sidx[p], semi[p]).wait()
    pltpu.make_async_copy(src_hbm.at[pl.ds(0, K)], didx[p], semi[p]).wait()

  def start_g(p):
    if mode == "feat":
      @pl.when(c == 0)
      def _():
        pltpu.async_copy(ha_hbm.at[sidx[p]], r_v, sem_g)

      @pl.when(c == 1)
      def _():
        pltpu.async_copy(hb_hbm.at[sidx[p]], r_v, sem_g)
    else:
      pltpu.async_copy(ha_hbm.at[sidx[p]], r_v, sem_g)

  def wait_g(p):
    pltpu.make_async_copy(ha_hbm.at[sidx[p]], r_v, sem_g).wait()

  def comp_ex(b, p):
    _edge_weights(ebase(b), mxval, als_v, ald_v, sidx[p], didx[p], exv[p])

  def scale(p):
    ev = exv[p]

    def _sc(k, _):
      exk = plsc.load_gather(ev, [jnp.full((16,), 0, jnp.int32) + k])
      for cc in range(d // 16):
        r_v[k, pl.ds(cc * 16, 16)] = r_v[k, pl.ds(cc * 16, 16)] * exk
      return 0
    lax.fori_loop(0, K, _sc, 0)

  def fire_s(p):
    pltpu.sync_copy(r_v, acc_s.at[didx[p]], add=True)

  def fire_d(p):
    if mode == "feat":
      @pl.when(c == 0)
      def _():
        pltpu.sync_copy(exv[p], den_s.at[didx[p]], add=True)
    else:
      pltpu.sync_copy(exv[p], den_s.at[didx[p]], add=True)

  fire_i(0, 0)
  fire_i(1, 1)

  pltpu.sync_copy(als_hbm, als_v)
  pltpu.sync_copy(ald_hbm, ald_v)
  pltpu.sync_copy(mx_hbm, mx_v)

  def _zrows(i, _):
    for cc in range(d // 16):
      r_v[i, pl.ds(cc * 16, 16)] = jnp.zeros((16,), jnp.float32)
    return 0
  lax.fori_loop(0, K, _zrows, 0)

  def _zden(i, _):
    zden_v[pl.ds(i * 16, 16)] = jnp.zeros((16,), jnp.float32)
    return 0
  lax.fori_loop(0, TR // 16, _zden, 0)

  for j in range(TR // K):
    pltpu.sync_copy(r_v, acc_s.at[pl.ds(s * TR + j * K, K)])
  pltpu.sync_copy(zden_v, den_s.at[pl.ds(s * TR, TR)])
  plsc.subcore_barrier()

  mxval = mx_v[pl.ds(0, 16)]

  def _half(b, p):
    wait_i(p)
    start_g(p)
    comp_ex(b, p)
    wait_g(p)
    scale(p)
    fire_s(p)
    fire_d(p)

    @pl.when(b + 2 < nb)
    def _():
      fire_i(b + 2, p)

  def _pair(i, _):
    b = 2 * i
    _half(b, 0)
    _half(b + 1, 1)
    return 0

  lax.fori_loop(0, nb // 2, _pair, 0)
  plsc.subcore_barrier()

  pltpu.sync_copy(acc_s.at[pl.ds(s * TR, TR)],
                  acc_hbm.at[c, pl.ds(s * TR, TR)])
  if mode == "feat":
    @pl.when(c == 0)
    def _():
      pltpu.sync_copy(den_s.at[pl.ds(s * TR, TR)],
                      den_hbm.at[pl.ds(s * TR, TR)])
  else:
    pltpu.sync_copy(den_s.at[pl.ds(s * TR, TR)],
                    den_hbm.at[c, pl.ds(s * TR, TR)])


def _sc_agg(mode, ha, hb, als, ald, mxv, src, dst):
  d = ha.shape[1]
  mesh = plsc.VectorSubcoreMesh(core_axis_name="c", subcore_axis_name="s",
                                num_cores=2, num_subcores=NSUB)
  if mode == "feat":
    den_t = jax.ShapeDtypeStruct((NP,), jnp.float32)
  else:
    den_t = jax.ShapeDtypeStruct((2, NP), jnp.float32)
  fn = pl.kernel(
      functools.partial(_sc_pipe_body, mode),
      out_type=[
          jax.ShapeDtypeStruct((2, NP, d), jnp.float32),
          den_t,
      ],
      mesh=mesh,
      compiler_params=pltpu.CompilerParams(needs_layout_passes=False),
      scratch_types=[
          pltpu.VMEM((NP,), jnp.float32),
          pltpu.VMEM((NP,), jnp.float32),
          pltpu.VMEM((16,), jnp.float32),
          pltpu.VMEM((K,), jnp.int32),
          pltpu.VMEM((K,), jnp.int32),
          pltpu.VMEM((K,), jnp.int32),
          pltpu.VMEM((K,), jnp.int32),
          pltpu.VMEM((K,), jnp.float32),
          pltpu.VMEM((K,), jnp.float32),
          pltpu.VMEM((K, d), jnp.float32),
          pltpu.VMEM((TR,), jnp.float32),
          pltpu.VMEM_SHARED((NP, d), jnp.float32),
          pltpu.VMEM_SHARED((NP,), jnp.float32),
          pltpu.SemaphoreType.DMA,
          pltpu.SemaphoreType.DMA,
          pltpu.SemaphoreType.DMA,
      ],
  )
  return fn(ha, hb, als, ald, mxv, src, dst)


def _sc_final_body(h3_hbm, als_hbm, ald_hbm, mx_hbm, src_hbm, dst_hbm,
                   num_hbm, den_hbm,
                   h3_v, als_v, ald_v, mx_v, s0_v, s1_v, d0_v, d1_v,
                   e0_v, e1_v, v0_v, v1_v, zden_v, num_s, den_s,
                   sem_i0, sem_i1, sem_s):
  c = lax.axis_index("c")
  s = lax.axis_index("s")
  sidx = (s0_v, s1_v)
  didx = (d0_v, d1_v)
  exv = (e0_v, e1_v)
  valv = (v0_v, v1_v)
  semi = (sem_i0, sem_i1)
  ept = EP // 32
  nb = ept // K
  ebase = lambda b: (s * 2 + c) * ept + b * K

  def fire_i(b, p):
    base = ebase(b)
    pltpu.async_copy(src_hbm.at[pl.ds(base, K)], sidx[p], semi[p])
    pltpu.async_copy(dst_hbm.at[pl.ds(base, K)], didx[p], semi[p])

  def wait_i(p):
    pltpu.make_async_copy(src_hbm.at[pl.ds(0, K)], sidx[p], semi[p]).wait()
    pltpu.make_async_copy(src_hbm.at[pl.ds(0, K)], didx[p], semi[p]).wait()

  def comp(b, p):
    base = ebase(b)
    for j in range(K // 16):
      si = sidx[p][pl.ds(j * 16, 16)]
      di = didx[p][pl.ds(j * 16, 16)]
      asg = plsc.load_gather(als_v, [si])
      adg = plsc.load_gather(ald_v, [di])
      e = _lrelu(asg + adg, 0.2)
      m = _lrelu(mxval + adg, 0.2)
      ex = jnp.exp(e - m)
      gi = base + (j * 16) + _iota16()
      ex = jnp.where(gi < E_REAL, ex, 0.0)
      h3g = plsc.load_gather(h3_v, [si])
      exv[p][pl.ds(j * 16, 16)] = ex
      valv[p][pl.ds(j * 16, 16)] = ex * h3g

  def fire_s(p):
    pltpu.sync_copy(valv[p], num_s.at[didx[p]], add=True)
    pltpu.sync_copy(exv[p], den_s.at[didx[p]], add=True)

  fire_i(0, 0)
  fire_i(1, 1)

  pltpu.sync_copy(h3_hbm, h3_v)
  pltpu.sync_copy(als_hbm, als_v)
  pltpu.sync_copy(ald_hbm, ald_v)
  pltpu.sync_copy(mx_hbm, mx_v)

  def _zden(i, _):
    zden_v[pl.ds(i * 16, 16)] = jnp.zeros((16,), jnp.float32)
    return 0
  lax.fori_loop(0, TR // 16, _zden, 0)
  pltpu.sync_copy(zden_v, num_s.at[pl.ds(s * TR, TR)])
  pltpu.sync_copy(zden_v, den_s.at[pl.ds(s * TR, TR)])
  plsc.subcore_barrier()

  mxval = mx_v[pl.ds(0, 16)]

  def _pair(i, _):
    b = 2 * i
    wait_i(0)
    comp(b, 0)
    fire_s(0)

    @pl.when(b + 2 < nb)
    def _():
      fire_i(b + 2, 0)

    b1 = b + 1
    wait_i(1)
    comp(b1, 1)
    fire_s(1)

    @pl.when(b1 + 2 < nb)
    def _():
      fire_i(b1 + 2, 1)

    return 0

  lax.fori_loop(0, nb // 2, _pair, 0)
  plsc.subcore_barrier()

  pltpu.sync_copy(num_s.at[pl.ds(s * TR, TR)],
                  num_hbm.at[c, pl.ds(s * TR, TR)])
  pltpu.sync_copy(den_s.at[pl.ds(s * TR, TR)],
                  den_hbm.at[c, pl.ds(s * TR, TR)])


def _sc_final(h3, als, ald, mxv, src, dst):
  mesh = plsc.VectorSubcoreMesh(core_axis_name="c", subcore_axis_name="s",
                                num_cores=2, num_subcores=NSUB)
  fn = pl.kernel(
      _sc_final_body,
      out_type=[
          jax.ShapeDtypeStruct((2, NP), jnp.float32),
          jax.ShapeDtypeStruct((2, NP), jnp.float32),
      ],
      mesh=mesh,
      compiler_params=pltpu.CompilerParams(needs_layout_passes=False),
      scratch_types=[
          pltpu.VMEM((NP,), jnp.float32),
          pltpu.VMEM((NP,), jnp.float32),
          pltpu.VMEM((NP,), jnp.float32),
          pltpu.VMEM((16,), jnp.float32),
          pltpu.VMEM((K,), jnp.int32),
          pltpu.VMEM((K,), jnp.int32),
          pltpu.VMEM((K,), jnp.int32),
          pltpu.VMEM((K,), jnp.int32),
          pltpu.VMEM((K,), jnp.float32),
          pltpu.VMEM((K,), jnp.float32),
          pltpu.VMEM((K,), jnp.float32),
          pltpu.VMEM((K,), jnp.float32),
          pltpu.VMEM((TR,), jnp.float32),
          pltpu.VMEM_SHARED((NP,), jnp.float32),
          pltpu.VMEM_SHARED((NP,), jnp.float32),
          pltpu.SemaphoreType.DMA,
          pltpu.SemaphoreType.DMA,
          pltpu.SemaphoreType.DMA,
      ],
  )
  return fn(h3, als, ald, mxv, src, dst)


def kernel(x, edge_index, W1, a_src1, a_dst1, b1,
           W2, a_src2, a_dst2, b2, W3, a_src3, a_dst3, b3):
  f32 = jnp.float32
  xp = jnp.pad(x.astype(f32), ((0, NP - N), (0, 0)))
  loop = jnp.arange(N, dtype=jnp.int32)
  padi = jnp.zeros((EP - E_REAL,), jnp.int32)
  src = jnp.concatenate([edge_index[0].astype(jnp.int32), loop, padi])
  dst = jnp.concatenate([edge_index[1].astype(jnp.int32), loop, padi])

  h1, als1, ald1, mx1 = _tc_first(xp, W1, a_src1.reshape(-1, 1),
                                  a_dst1.reshape(-1, 1))
  mx1v = jnp.broadcast_to(mx1.reshape(1), (16,))
  d2 = h1.shape[1] // 2
  acc1, den1 = _sc_agg("feat", h1[:, :d2], h1[:, d2:], als1[:, 0],
                       ald1[:, 0], mx1v, src, dst)
  accf1 = jnp.concatenate([acc1[0], acc1[1]], axis=1)

  h2, als2, ald2, mx2 = _tc_mid(accf1, den1.reshape(-1, 1), b1.reshape(1, -1),
                                W2, a_src2.reshape(-1, 1), a_dst2.reshape(-1, 1))
  mx2v = jnp.broadcast_to(mx2.reshape(1), (16,))
  acc2, den2 = _sc_agg("edge", h2, h2, als2[:, 0], ald2[:, 0], mx2v, src, dst)

  h3, als3, ald3, mx3 = _tc_mid2(acc2[0], acc2[1],
                                 den2[0].reshape(-1, 1), den2[1].reshape(-1, 1),
                                 b2.reshape(1, -1),
                                 W3, a_src3.reshape(-1, 1), a_dst3.reshape(-1, 1))
  mx3v = jnp.broadcast_to(mx3.reshape(1), (16,))
  num3, den3 = _sc_final(h3[:, 0], als3[:, 0], ald3[:, 0], mx3v, src, dst)
  outp = _tc_fin(num3[0].reshape(-1, 1), num3[1].reshape(-1, 1),
                 den3[0].reshape(-1, 1), den3[1].reshape(-1, 1),
                 b3.reshape(1, 1))
  return outp[:N, 0]


# speedup vs baseline: 31.6319x; 1.0332x over previous
import functools

import jax
import jax.numpy as jnp
from jax import lax
from jax.experimental import pallas as pl
from jax.experimental.pallas import tpu as pltpu
from jax.experimental.pallas import tpu_sc as plsc

N = 10000
NP = 10240
TR = NP // 16
K = 128
KG = 64
NSUB = 16
E_REAL = 650000
EP = 655360
EPT = EP // NSUB
NB = EPT // K
BR = 2048


def _lrelu(x, s):
  return jnp.where(x >= 0, x, s * x)


def _tc_first_body(x_ref, w_ref, as_ref, ad_ref,
                   h_ref, als_ref, ald_ref, mx_ref):
  i = pl.program_id(0)
  h = jnp.dot(x_ref[...], w_ref[...], preferred_element_type=jnp.float32)
  h_ref[...] = h
  als = jnp.dot(h, as_ref[...], preferred_element_type=jnp.float32)
  ald = jnp.dot(h, ad_ref[...], preferred_element_type=jnp.float32)
  als_ref[...] = als
  ald_ref[...] = ald

  @pl.when(i == 0)
  def _():
    mx_ref[...] = jnp.full((1, 1), -1e30, jnp.float32)

  mx_ref[...] = jnp.maximum(mx_ref[...], jnp.max(als))


def _tc_first(x, w, a_s, a_d):
  din, dout = w.shape
  grid = NP // BR
  return pl.pallas_call(
      _tc_first_body,
      grid=(grid,),
      in_specs=[
          pl.BlockSpec((BR, din), lambda i: (i, 0)),
          pl.BlockSpec((din, dout), lambda i: (0, 0)),
          pl.BlockSpec((dout, 1), lambda i: (0, 0)),
          pl.BlockSpec((dout, 1), lambda i: (0, 0)),
      ],
      out_specs=[
          pl.BlockSpec((BR, dout), lambda i: (i, 0)),
          pl.BlockSpec((BR, 1), lambda i: (i, 0)),
          pl.BlockSpec((BR, 1), lambda i: (i, 0)),
          pl.BlockSpec((1, 1), lambda i: (0, 0)),
      ],
      out_shape=[
          jax.ShapeDtypeStruct((NP, dout), jnp.float32),
          jax.ShapeDtypeStruct((NP, 1), jnp.float32),
          jax.ShapeDtypeStruct((NP, 1), jnp.float32),
          jax.ShapeDtypeStruct((1, 1), jnp.float32),
      ],
  )(x, w, a_s, a_d)


def _tc_mid_body(acc_ref, den_ref, b_ref, w_ref, as_ref, ad_ref,
                 h_ref, als_ref, ald_ref, mx_ref):
  i = pl.program_id(0)
  g = _lrelu(acc_ref[...] / (den_ref[...] + 1e-16) + b_ref[...], 0.01)
  row = i * BR + lax.broadcasted_iota(jnp.int32, (BR, 1), 0)
  g = jnp.where(row < N, g, 0.0)
  h = jnp.dot(g, w_ref[...], preferred_element_type=jnp.float32)
  h_ref[...] = h
  als = jnp.dot(h, as_ref[...], preferred_element_type=jnp.float32)
  ald = jnp.dot(h, ad_ref[...], preferred_element_type=jnp.float32)
  als_ref[...] = als
  ald_ref[...] = ald

  @pl.when(i == 0)
  def _():
    mx_ref[...] = jnp.full((1, 1), -1e30, jnp.float32)

  mx_ref[...] = jnp.maximum(mx_ref[...], jnp.max(als))


def _tc_mid(acc, den, b, w, a_s, a_d):
  din, dout = w.shape
  grid = NP // BR
  return pl.pallas_call(
      _tc_mid_body,
      grid=(grid,),
      in_specs=[
          pl.BlockSpec((BR, din), lambda i: (i, 0)),
          pl.BlockSpec((BR, 1), lambda i: (i, 0)),
          pl.BlockSpec((1, din), lambda i: (0, 0)),
          pl.BlockSpec((din, dout), lambda i: (0, 0)),
          pl.BlockSpec((dout, 1), lambda i: (0, 0)),
          pl.BlockSpec((dout, 1), lambda i: (0, 0)),
      ],
      out_specs=[
          pl.BlockSpec((BR, dout), lambda i: (i, 0)),
          pl.BlockSpec((BR, 1), lambda i: (i, 0)),
          pl.BlockSpec((BR, 1), lambda i: (i, 0)),
          pl.BlockSpec((1, 1), lambda i: (0, 0)),
      ],
      out_shape=[
          jax.ShapeDtypeStruct((NP, dout), jnp.float32),
          jax.ShapeDtypeStruct((NP, 1), jnp.float32),
          jax.ShapeDtypeStruct((NP, 1), jnp.float32),
          jax.ShapeDtypeStruct((1, 1), jnp.float32),
      ],
  )(acc, den, b, w, a_s, a_d)


def _tc_mid2_body(a0_ref, a1_ref, d0_ref, d1_ref, b_ref, w_ref, as_ref, ad_ref,
                  h_ref, als_ref, ald_ref, mx_ref):
  i = pl.program_id(0)
  acc = a0_ref[...] + a1_ref[...]
  den = d0_ref[...] + d1_ref[...]
  g = _lrelu(acc / (den + 1e-16) + b_ref[...], 0.01)
  row = i * BR + lax.broadcasted_iota(jnp.int32, (BR, 1), 0)
  g = jnp.where(row < N, g, 0.0)
  h = jnp.dot(g, w_ref[...], preferred_element_type=jnp.float32)
  h_ref[...] = h
  als = jnp.dot(h, as_ref[...], preferred_element_type=jnp.float32)
  ald = jnp.dot(h, ad_ref[...], preferred_element_type=jnp.float32)
  als_ref[...] = als
  ald_ref[...] = ald

  @pl.when(i == 0)
  def _():
    mx_ref[...] = jnp.full((1, 1), -1e30, jnp.float32)

  mx_ref[...] = jnp.maximum(mx_ref[...], jnp.max(als))


def _tc_mid2(a0, a1, d0, d1, b, w, a_s, a_d):
  din, dout = w.shape
  grid = NP // BR
  return pl.pallas_call(
      _tc_mid2_body,
      grid=(grid,),
      in_specs=[
          pl.BlockSpec((BR, din), lambda i: (i, 0)),
          pl.BlockSpec((BR, din), lambda i: (i, 0)),
          pl.BlockSpec((BR, 1), lambda i: (i, 0)),
          pl.BlockSpec((BR, 1), lambda i: (i, 0)),
          pl.BlockSpec((1, din), lambda i: (0, 0)),
          pl.BlockSpec((din, dout), lambda i: (0, 0)),
          pl.BlockSpec((dout, 1), lambda i: (0, 0)),
          pl.BlockSpec((dout, 1), lambda i: (0, 0)),
      ],
      out_specs=[
          pl.BlockSpec((BR, dout), lambda i: (i, 0)),
          pl.BlockSpec((BR, 1), lambda i: (i, 0)),
          pl.BlockSpec((BR, 1), lambda i: (i, 0)),
          pl.BlockSpec((1, 1), lambda i: (0, 0)),
      ],
      out_shape=[
          jax.ShapeDtypeStruct((NP, dout), jnp.float32),
          jax.ShapeDtypeStruct((NP, 1), jnp.float32),
          jax.ShapeDtypeStruct((NP, 1), jnp.float32),
          jax.ShapeDtypeStruct((1, 1), jnp.float32),
      ],
  )(a0, a1, d0, d1, b, w, a_s, a_d)


def _tc_fin_body(n0_ref, n1_ref, d0_ref, d1_ref, b_ref, o_ref):
  num = n0_ref[...] + n1_ref[...]
  den = d0_ref[...] + d1_ref[...]
  o_ref[...] = num / (den + 1e-16) + b_ref[...]


def _tc_fin(n0, n1, d0, d1, b3):
  grid = NP // BR
  return pl.pallas_call(
      _tc_fin_body,
      grid=(grid,),
      in_specs=[
          pl.BlockSpec((BR, 1), lambda i: (i, 0)),
          pl.BlockSpec((BR, 1), lambda i: (i, 0)),
          pl.BlockSpec((BR, 1), lambda i: (i, 0)),
          pl.BlockSpec((BR, 1), lambda i: (i, 0)),
          pl.BlockSpec((1, 1), lambda i: (0, 0)),
      ],
      out_specs=pl.BlockSpec((BR, 1), lambda i: (i, 0)),
      out_shape=jax.ShapeDtypeStruct((NP, 1), jnp.float32),
  )(n0, n1, d0, d1, b3)


def _iota16():
  return lax.iota(jnp.int32, 16)


def _edge_weights(base, mxval, als_v, ald_v, sidx_v, didx_v, ex_v):
  for j in range(KG // 16):
    si = sidx_v[pl.ds(j * 16, 16)]
    di = didx_v[pl.ds(j * 16, 16)]
    asg = plsc.load_gather(als_v, [si])
    adg = plsc.load_gather(ald_v, [di])
    e = _lrelu(asg + adg, 0.2)
    m = _lrelu(mxval + adg, 0.2)
    ex = jnp.exp(e - m)
    gi = base + (j * 16) + _iota16()
    ex = jnp.where(gi < E_REAL, ex, 0.0)
    ex_v[pl.ds(j * 16, 16)] = ex


def _sc_pipe_body(mode, ha_hbm, hb_hbm, als_hbm, ald_hbm, mx_hbm, src_hbm,
                  dst_hbm, acc_hbm, den_hbm,
                  als_v, ald_v, mx_v, s0_v, s1_v, d0_v, d1_v, c0_v, c1_v,
                  e0_v, e1_v, r0_v, r1_v, zden_v, acc_s, den_s,
                  sem_i0, sem_i1, sem_g, sem_s0, sem_s1):
  d = r0_v.shape[1]
  c = lax.axis_index("c")
  s = lax.axis_index("s")
  sidx = (s0_v, s1_v)
  didx = (d0_v, d1_v)
  dsc = (c0_v, c1_v)
  exv = (e0_v, e1_v)
  rows = (r0_v, r1_v)
  semi = (sem_i0, sem_i1)
  sems = (sem_s0, sem_s1)

  if mode == "feat":
    ept = EPT
    nb = ept // KG
    ebase = lambda b: s * ept + b * KG
  else:
    ept = EP // 32
    nb = ept // KG
    ebase = lambda b: (s * 2 + c) * ept + b * KG

  def fire_i(b, p):
    base = ebase(b)
    pltpu.async_copy(src_hbm.at[pl.ds(base, KG)], sidx[p], semi[p])
    pltpu.async_copy(dst_hbm.at[pl.ds(base, KG)], didx[p], semi[p])

  def wait_i(p):
    pltpu.make_async_copy(src_hbm.at[pl.ds(0, KG)], sidx[p], semi[p]).wait()
    pltpu.make_async_copy(src_hbm.at[pl.ds(0, KG)], didx[p], semi[p]).wait()

  def start_g(p):
    if mode == "feat":
      @pl.when(c == 0)
      def _():
        pltpu.async_copy(ha_hbm.at[sidx[p]], rows[p], sem_g)

      @pl.when(c == 1)
      def _():
        pltpu.async_copy(hb_hbm.at[sidx[p]], rows[p], sem_g)
    else:
      pltpu.async_copy(ha_hbm.at[sidx[p]], rows[p], sem_g)

  def wait_g(p):
    pltpu.make_async_copy(ha_hbm.at[sidx[p]], rows[p], sem_g).wait()

  def comp_ex(b, p):
    _edge_weights(ebase(b), mxval, als_v, ald_v, sidx[p], didx[p], exv[p])

  def save_didx(p):
    for j in range(KG // 16):
      dsc[p][pl.ds(j * 16, 16)] = didx[p][pl.ds(j * 16, 16)]

  def scale(p):
    rv = rows[p]
    ev = exv[p]

    def _sc(k, _):
      exk = plsc.load_gather(ev, [jnp.full((16,), 0, jnp.int32) + k])
      for cc in range(d // 16):
        rv[k, pl.ds(cc * 16, 16)] = rv[k, pl.ds(cc * 16, 16)] * exk
      return 0
    lax.fori_loop(0, KG, _sc, 0)

  def fire_s(p):
    pltpu.async_copy(rows[p], acc_s.at[dsc[p]], sems[p], add=True)
    if mode == "feat":
      @pl.when(c == 0)
      def _():
        pltpu.async_copy(exv[p], den_s.at[dsc[p]], sems[p], add=True)
    else:
      pltpu.async_copy(exv[p], den_s.at[dsc[p]], sems[p], add=True)

  def wait_s(p):
    pltpu.make_async_copy(rows[p], acc_s.at[dsc[p]], sems[p]).wait()
    if mode == "feat":
      @pl.when(c == 0)
      def _():
        pltpu.make_async_copy(exv[p], den_s.at[dsc[p]], sems[p]).wait()
    else:
      pltpu.make_async_copy(exv[p], den_s.at[dsc[p]], sems[p]).wait()

  fire_i(0, 0)
  fire_i(1, 1)

  pltpu.sync_copy(als_hbm, als_v)
  pltpu.sync_copy(ald_hbm, ald_v)
  pltpu.sync_copy(mx_hbm, mx_v)

  def _zrows(i, _):
    for cc in range(d // 16):
      r0_v[i, pl.ds(cc * 16, 16)] = jnp.zeros((16,), jnp.float32)
    return 0
  lax.fori_loop(0, KG, _zrows, 0)

  def _zden(i, _):
    zden_v[pl.ds(i * 16, 16)] = jnp.zeros((16,), jnp.float32)
    return 0
  lax.fori_loop(0, TR // 16, _zden, 0)

  for j in range(TR // KG):
    pltpu.sync_copy(r0_v, acc_s.at[pl.ds(s * TR + j * KG, KG)])
  pltpu.sync_copy(zden_v, den_s.at[pl.ds(s * TR, TR)])
  plsc.subcore_barrier()

  mxval = mx_v[pl.ds(0, 16)]

  def _half(b, p):
    wait_i(p)

    @pl.when(b >= 2)
    def _():
      wait_s(p)

    start_g(p)
    comp_ex(b, p)
    save_didx(p)
    wait_g(p)

    @pl.when(b + 2 < nb)
    def _():
      fire_i(b + 2, p)

    scale(p)
    fire_s(p)

  def _pair(i, _):
    b = 2 * i
    _half(b, 0)
    _half(b + 1, 1)
    return 0

  lax.fori_loop(0, nb // 2, _pair, 0)
  wait_s(0)
  wait_s(1)
  plsc.subcore_barrier()

  pltpu.sync_copy(acc_s.at[pl.ds(s * TR, TR)],
                  acc_hbm.at[c, pl.ds(s * TR, TR)])
  if mode == "feat":
    @pl.when(c == 0)
    def _():
      pltpu.sync_copy(den_s.at[pl.ds(s * TR, TR)],
                      den_hbm.at[pl.ds(s * TR, TR)])
  else:
    pltpu.sync_copy(den_s.at[pl.ds(s * TR, TR)],
                    den_hbm.at[c, pl.ds(s * TR, TR)])


def _sc_agg(mode, ha, hb, als, ald, mxv, src, dst):
  d = ha.shape[1]
  mesh = plsc.VectorSubcoreMesh(core_axis_name="c", subcore_axis_name="s",
                                num_cores=2, num_subcores=NSUB)
  if mode == "feat":
    den_t = jax.ShapeDtypeStruct((NP,), jnp.float32)
  else:
    den_t = jax.ShapeDtypeStruct((2, NP), jnp.float32)
  fn = pl.kernel(
      functools.partial(_sc_pipe_body, mode),
      out_type=[
          jax.ShapeDtypeStruct((2, NP, d), jnp.float32),
          den_t,
      ],
      mesh=mesh,
      compiler_params=pltpu.CompilerParams(needs_layout_passes=False),
      scratch_types=[
          pltpu.VMEM((NP,), jnp.float32),
          pltpu.VMEM((NP,), jnp.float32),
          pltpu.VMEM((16,), jnp.float32),
          pltpu.VMEM((KG,), jnp.int32),
          pltpu.VMEM((KG,), jnp.int32),
          pltpu.VMEM((KG,), jnp.int32),
          pltpu.VMEM((KG,), jnp.int32),
          pltpu.VMEM((KG,), jnp.int32),
          pltpu.VMEM((KG,), jnp.int32),
          pltpu.VMEM((KG,), jnp.float32),
          pltpu.VMEM((KG,), jnp.float32),
          pltpu.VMEM((KG, d), jnp.float32),
          pltpu.VMEM((KG, d), jnp.float32),
          pltpu.VMEM((TR,), jnp.float32),
          pltpu.VMEM_SHARED((NP, d), jnp.float32),
          pltpu.VMEM_SHARED((NP,), jnp.float32),
          pltpu.SemaphoreType.DMA,
          pltpu.SemaphoreType.DMA,
          pltpu.SemaphoreType.DMA,
          pltpu.SemaphoreType.DMA,
          pltpu.SemaphoreType.DMA,
      ],
  )
  return fn(ha, hb, als, ald, mxv, src, dst)


def _sc_final_body(h3_hbm, als_hbm, ald_hbm, mx_hbm, src_hbm, dst_hbm,
                   num_hbm, den_hbm,
                   h3_v, als_v, ald_v, mx_v, s0_v, s1_v, d0_v, d1_v,
                   e0_v, e1_v, v0_v, v1_v, zden_v, num_s, den_s,
                   sem_i0, sem_i1, sem_s):
  c = lax.axis_index("c")
  s = lax.axis_index("s")
  sidx = (s0_v, s1_v)
  didx = (d0_v, d1_v)
  exv = (e0_v, e1_v)
  valv = (v0_v, v1_v)
  semi = (sem_i0, sem_i1)
  ept = EP // 32
  nb = ept // K
  ebase = lambda b: (s * 2 + c) * ept + b * K

  def fire_i(b, p):
    base = ebase(b)
    pltpu.async_copy(src_hbm.at[pl.ds(base, K)], sidx[p], semi[p])
    pltpu.async_copy(dst_hbm.at[pl.ds(base, K)], didx[p], semi[p])

  def wait_i(p):
    pltpu.make_async_copy(src_hbm.at[pl.ds(0, K)], sidx[p], semi[p]).wait()
    pltpu.make_async_copy(src_hbm.at[pl.ds(0, K)], didx[p], semi[p]).wait()

  def comp(b, p):
    base = ebase(b)
    for j in range(K // 16):
      si = sidx[p][pl.ds(j * 16, 16)]
      di = didx[p][pl.ds(j * 16, 16)]
      asg = plsc.load_gather(als_v, [si])
      adg = plsc.load_gather(ald_v, [di])
      e = _lrelu(asg + adg, 0.2)
      m = _lrelu(mxval + adg, 0.2)
      ex = jnp.exp(e - m)
      gi = base + (j * 16) + _iota16()
      ex = jnp.where(gi < E_REAL, ex, 0.0)
      h3g = plsc.load_gather(h3_v, [si])
      exv[p][pl.ds(j * 16, 16)] = ex
      valv[p][pl.ds(j * 16, 16)] = ex * h3g

  def fire_s(p):
    pltpu.sync_copy(valv[p], num_s.at[didx[p]], add=True)
    pltpu.sync_copy(exv[p], den_s.at[didx[p]], add=True)

  fire_i(0, 0)
  fire_i(1, 1)

  pltpu.sync_copy(h3_hbm, h3_v)
  pltpu.sync_copy(als_hbm, als_v)
  pltpu.sync_copy(ald_hbm, ald_v)
  pltpu.sync_copy(mx_hbm, mx_v)

  def _zden(i, _):
    zden_v[pl.ds(i * 16, 16)] = jnp.zeros((16,), jnp.float32)
    return 0
  lax.fori_loop(0, TR // 16, _zden, 0)
  pltpu.sync_copy(zden_v, num_s.at[pl.ds(s * TR, TR)])
  pltpu.sync_copy(zden_v, den_s.at[pl.ds(s * TR, TR)])
  plsc.subcore_barrier()

  mxval = mx_v[pl.ds(0, 16)]

  def _pair(i, _):
    b = 2 * i
    wait_i(0)
    comp(b, 0)
    fire_s(0)

    @pl.when(b + 2 < nb)
    def _():
      fire_i(b + 2, 0)

    b1 = b + 1
    wait_i(1)
    comp(b1, 1)
    fire_s(1)

    @pl.when(b1 + 2 < nb)
    def _():
      fire_i(b1 + 2, 1)

    return 0

  lax.fori_loop(0, nb // 2, _pair, 0)
  plsc.subcore_barrier()

  pltpu.sync_copy(num_s.at[pl.ds(s * TR, TR)],
                  num_hbm.at[c, pl.ds(s * TR, TR)])
  pltpu.sync_copy(den_s.at[pl.ds(s * TR, TR)],
                  den_hbm.at[c, pl.ds(s * TR, TR)])


def _sc_final(h3, als, ald, mxv, src, dst):
  mesh = plsc.VectorSubcoreMesh(core_axis_name="c", subcore_axis_name="s",
                                num_cores=2, num_subcores=NSUB)
  fn = pl.kernel(
      _sc_final_body,
      out_type=[
          jax.ShapeDtypeStruct((2, NP), jnp.float32),
          jax.ShapeDtypeStruct((2, NP), jnp.float32),
      ],
      mesh=mesh,
      compiler_params=pltpu.CompilerParams(needs_layout_passes=False),
      scratch_types=[
          pltpu.VMEM((NP,), jnp.float32),
          pltpu.VMEM((NP,), jnp.float32),
          pltpu.VMEM((NP,), jnp.float32),
          pltpu.VMEM((16,), jnp.float32),
          pltpu.VMEM((K,), jnp.int32),
          pltpu.VMEM((K,), jnp.int32),
          pltpu.VMEM((K,), jnp.int32),
          pltpu.VMEM((K,), jnp.int32),
          pltpu.VMEM((K,), jnp.float32),
          pltpu.VMEM((K,), jnp.float32),
          pltpu.VMEM((K,), jnp.float32),
          pltpu.VMEM((K,), jnp.float32),
          pltpu.VMEM((TR,), jnp.float32),
          pltpu.VMEM_SHARED((NP,), jnp.float32),
          pltpu.VMEM_SHARED((NP,), jnp.float32),
          pltpu.SemaphoreType.DMA,
          pltpu.SemaphoreType.DMA,
          pltpu.SemaphoreType.DMA,
      ],
  )
  return fn(h3, als, ald, mxv, src, dst)


def kernel(x, edge_index, W1, a_src1, a_dst1, b1,
           W2, a_src2, a_dst2, b2, W3, a_src3, a_dst3, b3):
  f32 = jnp.float32
  xp = jnp.pad(x.astype(f32), ((0, NP - N), (0, 0)))
  loop = jnp.arange(N, dtype=jnp.int32)
  padi = jnp.zeros((EP - E_REAL,), jnp.int32)
  src = jnp.concatenate([edge_index[0].astype(jnp.int32), loop, padi])
  dst = jnp.concatenate([edge_index[1].astype(jnp.int32), loop, padi])

  h1, als1, ald1, mx1 = _tc_first(xp, W1, a_src1.reshape(-1, 1),
                                  a_dst1.reshape(-1, 1))
  mx1v = jnp.broadcast_to(mx1.reshape(1), (16,))
  d2 = h1.shape[1] // 2
  acc1, den1 = _sc_agg("feat", h1[:, :d2], h1[:, d2:], als1[:, 0],
                       ald1[:, 0], mx1v, src, dst)
  accf1 = jnp.concatenate([acc1[0], acc1[1]], axis=1)

  h2, als2, ald2, mx2 = _tc_mid(accf1, den1.reshape(-1, 1), b1.reshape(1, -1),
                                W2, a_src2.reshape(-1, 1), a_dst2.reshape(-1, 1))
  mx2v = jnp.broadcast_to(mx2.reshape(1), (16,))
  acc2, den2 = _sc_agg("edge", h2, h2, als2[:, 0], ald2[:, 0], mx2v, src, dst)

  h3, als3, ald3, mx3 = _tc_mid2(acc2[0], acc2[1],
                                 den2[0].reshape(-1, 1), den2[1].reshape(-1, 1),
                                 b2.reshape(1, -1),
                                 W3, a_src3.reshape(-1, 1), a_dst3.reshape(-1, 1))
  mx3v = jnp.broadcast_to(mx3.reshape(1), (16,))
  num3, den3 = _sc_final(h3[:, 0], als3[:, 0], ald3[:, 0], mx3v, src, dst)
  outp = _tc_fin(num3[0].reshape(-1, 1), num3[1].reshape(-1, 1),
                 den3[0].reshape(-1, 1), den3[1].reshape(-1, 1),
                 b3.reshape(1, 1))
  return outp[:N, 0]


# speedup vs baseline: 32.5731x; 1.0298x over previous
import functools

import jax
import jax.numpy as jnp
from jax import lax
from jax.experimental import pallas as pl
from jax.experimental.pallas import tpu as pltpu
from jax.experimental.pallas import tpu_sc as plsc

N = 10000
NP = 10240
TR = NP // 16
K = 128
KG = 64
NSUB = 16
E_REAL = 650000
EP = 655360
EPT = EP // NSUB
NB = EPT // K
BR = 2048


def _lrelu(x, s):
  return jnp.where(x >= 0, x, s * x)


def _tc_first_body(x_ref, w_ref, as_ref, ad_ref,
                   h_ref, als_ref, ald_ref, mx_ref):
  i = pl.program_id(0)
  h = jnp.dot(x_ref[...], w_ref[...], preferred_element_type=jnp.float32)
  h_ref[...] = h
  als = jnp.dot(h, as_ref[...], preferred_element_type=jnp.float32)
  ald = jnp.dot(h, ad_ref[...], preferred_element_type=jnp.float32)
  als_ref[...] = als
  ald_ref[...] = ald

  @pl.when(i == 0)
  def _():
    mx_ref[...] = jnp.full((1, 1), -1e30, jnp.float32)

  mx_ref[...] = jnp.maximum(mx_ref[...], jnp.max(als))


def _tc_first(x, w, a_s, a_d):
  din, dout = w.shape
  grid = NP // BR
  return pl.pallas_call(
      _tc_first_body,
      grid=(grid,),
      in_specs=[
          pl.BlockSpec((BR, din), lambda i: (i, 0)),
          pl.BlockSpec((din, dout), lambda i: (0, 0)),
          pl.BlockSpec((dout, 1), lambda i: (0, 0)),
          pl.BlockSpec((dout, 1), lambda i: (0, 0)),
      ],
      out_specs=[
          pl.BlockSpec((BR, dout), lambda i: (i, 0)),
          pl.BlockSpec((BR, 1), lambda i: (i, 0)),
          pl.BlockSpec((BR, 1), lambda i: (i, 0)),
          pl.BlockSpec((1, 1), lambda i: (0, 0)),
      ],
      out_shape=[
          jax.ShapeDtypeStruct((NP, dout), jnp.float32),
          jax.ShapeDtypeStruct((NP, 1), jnp.float32),
          jax.ShapeDtypeStruct((NP, 1), jnp.float32),
          jax.ShapeDtypeStruct((1, 1), jnp.float32),
      ],
  )(x, w, a_s, a_d)


def _tc_mid_body(acc_ref, den_ref, b_ref, w_ref, as_ref, ad_ref,
                 h_ref, als_ref, ald_ref, mx_ref):
  i = pl.program_id(0)
  g = _lrelu(acc_ref[...] / (den_ref[...] + 1e-16) + b_ref[...], 0.01)
  row = i * BR + lax.broadcasted_iota(jnp.int32, (BR, 1), 0)
  g = jnp.where(row < N, g, 0.0)
  h = jnp.dot(g, w_ref[...], preferred_element_type=jnp.float32)
  h_ref[...] = h
  als = jnp.dot(h, as_ref[...], preferred_element_type=jnp.float32)
  ald = jnp.dot(h, ad_ref[...], preferred_element_type=jnp.float32)
  als_ref[...] = als
  ald_ref[...] = ald

  @pl.when(i == 0)
  def _():
    mx_ref[...] = jnp.full((1, 1), -1e30, jnp.float32)

  mx_ref[...] = jnp.maximum(mx_ref[...], jnp.max(als))


def _tc_mid(acc, den, b, w, a_s, a_d):
  din, dout = w.shape
  grid = NP // BR
  return pl.pallas_call(
      _tc_mid_body,
      grid=(grid,),
      in_specs=[
          pl.BlockSpec((BR, din), lambda i: (i, 0)),
          pl.BlockSpec((BR, 1), lambda i: (i, 0)),
          pl.BlockSpec((1, din), lambda i: (0, 0)),
          pl.BlockSpec((din, dout), lambda i: (0, 0)),
          pl.BlockSpec((dout, 1), lambda i: (0, 0)),
          pl.BlockSpec((dout, 1), lambda i: (0, 0)),
      ],
      out_specs=[
          pl.BlockSpec((BR, dout), lambda i: (i, 0)),
          pl.BlockSpec((BR, 1), lambda i: (i, 0)),
          pl.BlockSpec((BR, 1), lambda i: (i, 0)),
          pl.BlockSpec((1, 1), lambda i: (0, 0)),
      ],
      out_shape=[
          jax.ShapeDtypeStruct((NP, dout), jnp.float32),
          jax.ShapeDtypeStruct((NP, 1), jnp.float32),
          jax.ShapeDtypeStruct((NP, 1), jnp.float32),
          jax.ShapeDtypeStruct((1, 1), jnp.float32),
      ],
  )(acc, den, b, w, a_s, a_d)


def _tc_mid2_body(a0_ref, a1_ref, d0_ref, d1_ref, b_ref, w_ref, as_ref, ad_ref,
                  h_ref, als_ref, ald_ref, mx_ref):
  i = pl.program_id(0)
  acc = a0_ref[...] + a1_ref[...]
  den = d0_ref[...] + d1_ref[...]
  g = _lrelu(acc / (den + 1e-16) + b_ref[...], 0.01)
  row = i * BR + lax.broadcasted_iota(jnp.int32, (BR, 1), 0)
  g = jnp.where(row < N, g, 0.0)
  h = jnp.dot(g, w_ref[...], preferred_element_type=jnp.float32)
  h_ref[...] = h
  als = jnp.dot(h, as_ref[...], preferred_element_type=jnp.float32)
  ald = jnp.dot(h, ad_ref[...], preferred_element_type=jnp.float32)
  als_ref[...] = als
  ald_ref[...] = ald

  @pl.when(i == 0)
  def _():
    mx_ref[...] = jnp.full((1, 1), -1e30, jnp.float32)

  mx_ref[...] = jnp.maximum(mx_ref[...], jnp.max(als))


def _tc_mid2(a0, a1, d0, d1, b, w, a_s, a_d):
  din, dout = w.shape
  grid = NP // BR
  return pl.pallas_call(
      _tc_mid2_body,
      grid=(grid,),
      in_specs=[
          pl.BlockSpec((BR, din), lambda i: (i, 0)),
          pl.BlockSpec((BR, din), lambda i: (i, 0)),
          pl.BlockSpec((BR, 1), lambda i: (i, 0)),
          pl.BlockSpec((BR, 1), lambda i: (i, 0)),
          pl.BlockSpec((1, din), lambda i: (0, 0)),
          pl.BlockSpec((din, dout), lambda i: (0, 0)),
          pl.BlockSpec((dout, 1), lambda i: (0, 0)),
          pl.BlockSpec((dout, 1), lambda i: (0, 0)),
      ],
      out_specs=[
          pl.BlockSpec((BR, dout), lambda i: (i, 0)),
          pl.BlockSpec((BR, 1), lambda i: (i, 0)),
          pl.BlockSpec((BR, 1), lambda i: (i, 0)),
          pl.BlockSpec((1, 1), lambda i: (0, 0)),
      ],
      out_shape=[
          jax.ShapeDtypeStruct((NP, dout), jnp.float32),
          jax.ShapeDtypeStruct((NP, 1), jnp.float32),
          jax.ShapeDtypeStruct((NP, 1), jnp.float32),
          jax.ShapeDtypeStruct((1, 1), jnp.float32),
      ],
  )(a0, a1, d0, d1, b, w, a_s, a_d)


def _tc_fin_body(n0_ref, n1_ref, d0_ref, d1_ref, b_ref, o_ref):
  num = n0_ref[...] + n1_ref[...]
  den = d0_ref[...] + d1_ref[...]
  o_ref[...] = num / (den + 1e-16) + b_ref[...]


def _tc_fin(n0, n1, d0, d1, b3):
  grid = NP // BR
  return pl.pallas_call(
      _tc_fin_body,
      grid=(grid,),
      in_specs=[
          pl.BlockSpec((BR, 1), lambda i: (i, 0)),
          pl.BlockSpec((BR, 1), lambda i: (i, 0)),
          pl.BlockSpec((BR, 1), lambda i: (i, 0)),
          pl.BlockSpec((BR, 1), lambda i: (i, 0)),
          pl.BlockSpec((1, 1), lambda i: (0, 0)),
      ],
      out_specs=pl.BlockSpec((BR, 1), lambda i: (i, 0)),
      out_shape=jax.ShapeDtypeStruct((NP, 1), jnp.float32),
  )(n0, n1, d0, d1, b3)


def _iota16():
  return lax.iota(jnp.int32, 16)


def _edge_weights(base, mxval, als_v, ald_v, sidx_v, didx_v, ex_v):
  for j in range(KG // 16):
    si = sidx_v[pl.ds(j * 16, 16)]
    di = didx_v[pl.ds(j * 16, 16)]
    asg = plsc.load_gather(als_v, [si])
    adg = plsc.load_gather(ald_v, [di])
    e = _lrelu(asg + adg, 0.2)
    m = _lrelu(mxval + adg, 0.2)
    ex = jnp.exp(e - m)
    gi = base + (j * 16) + _iota16()
    ex = jnp.where(gi < E_REAL, ex, 0.0)
    ex_v[pl.ds(j * 16, 16)] = ex


def _sc_pipe_body(mode, ha_hbm, hb_hbm, als_hbm, ald_hbm, mx_hbm, src_hbm,
                  dst_hbm, acc_hbm, den_hbm,
                  als_v, ald_v, mx_v, s0_v, s1_v, d0_v, d1_v, c0_v, c1_v,
                  e0_v, e1_v, r0_v, r1_v, zden_v, acc_s, den_s,
                  sem_i0, sem_i1, sem_g, sem_s0, sem_s1):
  d = r0_v.shape[1]
  c = lax.axis_index("c")
  s = lax.axis_index("s")
  sidx = (s0_v, s1_v)
  didx = (d0_v, d1_v)
  dsc = (c0_v, c1_v)
  exv = (e0_v, e1_v)
  rows = (r0_v, r1_v)
  semi = (sem_i0, sem_i1)
  sems = (sem_s0, sem_s1)

  if mode == "feat":
    ept = EPT
    nb = ept // KG
    ebase = lambda b: s * ept + b * KG
  else:
    ept = EP // 32
    nb = ept // KG
    ebase = lambda b: (s * 2 + c) * ept + b * KG

  def fire_i(b, p):
    base = ebase(b)
    pltpu.async_copy(src_hbm.at[pl.ds(base, KG)], sidx[p], semi[p])
    pltpu.async_copy(dst_hbm.at[pl.ds(base, KG)], didx[p], semi[p])

  def wait_i(p):
    pltpu.make_async_copy(src_hbm.at[pl.ds(0, KG)], sidx[p], semi[p]).wait()
    pltpu.make_async_copy(src_hbm.at[pl.ds(0, KG)], didx[p], semi[p]).wait()

  def start_g(p):
    if mode == "feat":
      @pl.when(c == 0)
      def _():
        pltpu.async_copy(ha_hbm.at[sidx[p]], rows[p], sem_g)

      @pl.when(c == 1)
      def _():
        pltpu.async_copy(hb_hbm.at[sidx[p]], rows[p], sem_g)
    else:
      pltpu.async_copy(ha_hbm.at[sidx[p]], rows[p], sem_g)

  def wait_g(p):
    pltpu.make_async_copy(ha_hbm.at[sidx[p]], rows[p], sem_g).wait()

  def comp_ex(b, p):
    _edge_weights(ebase(b), mxval, als_v, ald_v, sidx[p], didx[p], exv[p])

  def save_didx(p):
    for j in range(KG // 16):
      dsc[p][pl.ds(j * 16, 16)] = didx[p][pl.ds(j * 16, 16)]

  def scale(p):
    rv = rows[p]
    ev = exv[p]

    def _sc(k4, _):
      for u in range(4):
        k = k4 * 4 + u
        exk = plsc.load_gather(ev, [jnp.full((16,), 0, jnp.int32) + k])
        for cc in range(d // 16):
          rv[k, pl.ds(cc * 16, 16)] = rv[k, pl.ds(cc * 16, 16)] * exk
      return 0
    lax.fori_loop(0, KG // 4, _sc, 0)

  def fire_s(p):
    pltpu.async_copy(rows[p], acc_s.at[dsc[p]], sems[p], add=True)
    if mode == "feat":
      @pl.when(c == 0)
      def _():
        pltpu.async_copy(exv[p], den_s.at[dsc[p]], sems[p], add=True)
    else:
      pltpu.async_copy(exv[p], den_s.at[dsc[p]], sems[p], add=True)

  def wait_s(p):
    pltpu.make_async_copy(rows[p], acc_s.at[dsc[p]], sems[p]).wait()
    if mode == "feat":
      @pl.when(c == 0)
      def _():
        pltpu.make_async_copy(exv[p], den_s.at[dsc[p]], sems[p]).wait()
    else:
      pltpu.make_async_copy(exv[p], den_s.at[dsc[p]], sems[p]).wait()

  fire_i(0, 0)
  fire_i(1, 1)

  pltpu.sync_copy(als_hbm, als_v)
  pltpu.sync_copy(ald_hbm, ald_v)
  pltpu.sync_copy(mx_hbm, mx_v)

  def _zrows(i, _):
    for cc in range(d // 16):
      r0_v[i, pl.ds(cc * 16, 16)] = jnp.zeros((16,), jnp.float32)
    return 0
  lax.fori_loop(0, KG, _zrows, 0)

  def _zden(i, _):
    zden_v[pl.ds(i * 16, 16)] = jnp.zeros((16,), jnp.float32)
    return 0
  lax.fori_loop(0, TR // 16, _zden, 0)

  for j in range(TR // KG):
    pltpu.sync_copy(r0_v, acc_s.at[pl.ds(s * TR + j * KG, KG)])
  pltpu.sync_copy(zden_v, den_s.at[pl.ds(s * TR, TR)])
  plsc.subcore_barrier()

  mxval = mx_v[pl.ds(0, 16)]

  def _half(b, p):
    wait_i(p)

    @pl.when(b >= 2)
    def _():
      wait_s(p)

    start_g(p)
    comp_ex(b, p)
    save_didx(p)
    wait_g(p)

    @pl.when(b + 2 < nb)
    def _():
      fire_i(b + 2, p)

    scale(p)
    fire_s(p)

  def _pair(i, _):
    b = 2 * i
    _half(b, 0)
    _half(b + 1, 1)
    return 0

  lax.fori_loop(0, nb // 2, _pair, 0)
  wait_s(0)
  wait_s(1)
  plsc.subcore_barrier()

  pltpu.sync_copy(acc_s.at[pl.ds(s * TR, TR)],
                  acc_hbm.at[c, pl.ds(s * TR, TR)])
  if mode == "feat":
    @pl.when(c == 0)
    def _():
      pltpu.sync_copy(den_s.at[pl.ds(s * TR, TR)],
                      den_hbm.at[pl.ds(s * TR, TR)])
  else:
    pltpu.sync_copy(den_s.at[pl.ds(s * TR, TR)],
                    den_hbm.at[c, pl.ds(s * TR, TR)])


def _sc_agg(mode, ha, hb, als, ald, mxv, src, dst):
  d = ha.shape[1]
  mesh = plsc.VectorSubcoreMesh(core_axis_name="c", subcore_axis_name="s",
                                num_cores=2, num_subcores=NSUB)
  if mode == "feat":
    den_t = jax.ShapeDtypeStruct((NP,), jnp.float32)
  else:
    den_t = jax.ShapeDtypeStruct((2, NP), jnp.float32)
  fn = pl.kernel(
      functools.partial(_sc_pipe_body, mode),
      out_type=[
          jax.ShapeDtypeStruct((2, NP, d), jnp.float32),
          den_t,
      ],
      mesh=mesh,
      compiler_params=pltpu.CompilerParams(needs_layout_passes=False),
      scratch_types=[
          pltpu.VMEM((NP,), jnp.float32),
          pltpu.VMEM((NP,), jnp.float32),
          pltpu.VMEM((16,), jnp.float32),
          pltpu.VMEM((KG,), jnp.int32),
          pltpu.VMEM((KG,), jnp.int32),
          pltpu.VMEM((KG,), jnp.int32),
          pltpu.VMEM((KG,), jnp.int32),
          pltpu.VMEM((KG,), jnp.int32),
          pltpu.VMEM((KG,), jnp.int32),
          pltpu.VMEM((KG,), jnp.float32),
          pltpu.VMEM((KG,), jnp.float32),
          pltpu.VMEM((KG, d), jnp.float32),
          pltpu.VMEM((KG, d), jnp.float32),
          pltpu.VMEM((TR,), jnp.float32),
          pltpu.VMEM_SHARED((NP, d), jnp.float32),
          pltpu.VMEM_SHARED((NP,), jnp.float32),
          pltpu.SemaphoreType.DMA,
          pltpu.SemaphoreType.DMA,
          pltpu.SemaphoreType.DMA,
          pltpu.SemaphoreType.DMA,
          pltpu.SemaphoreType.DMA,
      ],
  )
  return fn(ha, hb, als, ald, mxv, src, dst)


def _sc_final_body(h3_hbm, als_hbm, ald_hbm, mx_hbm, src_hbm, dst_hbm,
                   num_hbm, den_hbm,
                   h3_v, als_v, ald_v, mx_v, s0_v, s1_v, d0_v, d1_v,
                   e0_v, e1_v, v0_v, v1_v, zden_v, num_s, den_s,
                   sem_i0, sem_i1, sem_s):
  c = lax.axis_index("c")
  s = lax.axis_index("s")
  sidx = (s0_v, s1_v)
  didx = (d0_v, d1_v)
  exv = (e0_v, e1_v)
  valv = (v0_v, v1_v)
  semi = (sem_i0, sem_i1)
  ept = EP // 32
  nb = ept // K
  ebase = lambda b: (s * 2 + c) * ept + b * K

  def fire_i(b, p):
    base = ebase(b)
    pltpu.async_copy(src_hbm.at[pl.ds(base, K)], sidx[p], semi[p])
    pltpu.async_copy(dst_hbm.at[pl.ds(base, K)], didx[p], semi[p])

  def wait_i(p):
    pltpu.make_async_copy(src_hbm.at[pl.ds(0, K)], sidx[p], semi[p]).wait()
    pltpu.make_async_copy(src_hbm.at[pl.ds(0, K)], didx[p], semi[p]).wait()

  def comp(b, p):
    base = ebase(b)
    for j in range(K // 16):
      si = sidx[p][pl.ds(j * 16, 16)]
      di = didx[p][pl.ds(j * 16, 16)]
      asg = plsc.load_gather(als_v, [si])
      adg = plsc.load_gather(ald_v, [di])
      e = _lrelu(asg + adg, 0.2)
      m = _lrelu(mxval + adg, 0.2)
      ex = jnp.exp(e - m)
      gi = base + (j * 16) + _iota16()
      ex = jnp.where(gi < E_REAL, ex, 0.0)
      h3g = plsc.load_gather(h3_v, [si])
      exv[p][pl.ds(j * 16, 16)] = ex
      valv[p][pl.ds(j * 16, 16)] = ex * h3g

  def fire_s(p):
    pltpu.sync_copy(valv[p], num_s.at[didx[p]], add=True)
    pltpu.sync_copy(exv[p], den_s.at[didx[p]], add=True)

  fire_i(0, 0)
  fire_i(1, 1)

  pltpu.sync_copy(h3_hbm, h3_v)
  pltpu.sync_copy(als_hbm, als_v)
  pltpu.sync_copy(ald_hbm, ald_v)
  pltpu.sync_copy(mx_hbm, mx_v)

  def _zden(i, _):
    zden_v[pl.ds(i * 16, 16)] = jnp.zeros((16,), jnp.float32)
    return 0
  lax.fori_loop(0, TR // 16, _zden, 0)
  pltpu.sync_copy(zden_v, num_s.at[pl.ds(s * TR, TR)])
  pltpu.sync_copy(zden_v, den_s.at[pl.ds(s * TR, TR)])
  plsc.subcore_barrier()

  mxval = mx_v[pl.ds(0, 16)]

  def _pair(i, _):
    b = 2 * i
    wait_i(0)
    comp(b, 0)
    fire_s(0)

    @pl.when(b + 2 < nb)
    def _():
      fire_i(b + 2, 0)

    b1 = b + 1
    wait_i(1)
    comp(b1, 1)
    fire_s(1)

    @pl.when(b1 + 2 < nb)
    def _():
      fire_i(b1 + 2, 1)

    return 0

  lax.fori_loop(0, nb // 2, _pair, 0)
  plsc.subcore_barrier()

  pltpu.sync_copy(num_s.at[pl.ds(s * TR, TR)],
                  num_hbm.at[c, pl.ds(s * TR, TR)])
  pltpu.sync_copy(den_s.at[pl.ds(s * TR, TR)],
                  den_hbm.at[c, pl.ds(s * TR, TR)])


def _sc_final(h3, als, ald, mxv, src, dst):
  mesh = plsc.VectorSubcoreMesh(core_axis_name="c", subcore_axis_name="s",
                                num_cores=2, num_subcores=NSUB)
  fn = pl.kernel(
      _sc_final_body,
      out_type=[
          jax.ShapeDtypeStruct((2, NP), jnp.float32),
          jax.ShapeDtypeStruct((2, NP), jnp.float32),
      ],
      mesh=mesh,
      compiler_params=pltpu.CompilerParams(needs_layout_passes=False),
      scratch_types=[
          pltpu.VMEM((NP,), jnp.float32),
          pltpu.VMEM((NP,), jnp.float32),
          pltpu.VMEM((NP,), jnp.float32),
          pltpu.VMEM((16,), jnp.float32),
          pltpu.VMEM((K,), jnp.int32),
          pltpu.VMEM((K,), jnp.int32),
          pltpu.VMEM((K,), jnp.int32),
          pltpu.VMEM((K,), jnp.int32),
          pltpu.VMEM((K,), jnp.float32),
          pltpu.VMEM((K,), jnp.float32),
          pltpu.VMEM((K,), jnp.float32),
          pltpu.VMEM((K,), jnp.float32),
          pltpu.VMEM((TR,), jnp.float32),
          pltpu.VMEM_SHARED((NP,), jnp.float32),
          pltpu.VMEM_SHARED((NP,), jnp.float32),
          pltpu.SemaphoreType.DMA,
          pltpu.SemaphoreType.DMA,
          pltpu.SemaphoreType.DMA,
      ],
  )
  return fn(h3, als, ald, mxv, src, dst)


def kernel(x, edge_index, W1, a_src1, a_dst1, b1,
           W2, a_src2, a_dst2, b2, W3, a_src3, a_dst3, b3):
  f32 = jnp.float32
  xp = jnp.pad(x.astype(f32), ((0, NP - N), (0, 0)))
  loop = jnp.arange(N, dtype=jnp.int32)
  padi = jnp.zeros((EP - E_REAL,), jnp.int32)
  src = jnp.concatenate([edge_index[0].astype(jnp.int32), loop, padi])
  dst = jnp.concatenate([edge_index[1].astype(jnp.int32), loop, padi])

  h1, als1, ald1, mx1 = _tc_first(xp, W1, a_src1.reshape(-1, 1),
                                  a_dst1.reshape(-1, 1))
  mx1v = jnp.broadcast_to(mx1.reshape(1), (16,))
  d2 = h1.shape[1] // 2
  acc1, den1 = _sc_agg("feat", h1[:, :d2], h1[:, d2:], als1[:, 0],
                       ald1[:, 0], mx1v, src, dst)
  accf1 = jnp.concatenate([acc1[0], acc1[1]], axis=1)

  h2, als2, ald2, mx2 = _tc_mid(accf1, den1.reshape(-1, 1), b1.reshape(1, -1),
                                W2, a_src2.reshape(-1, 1), a_dst2.reshape(-1, 1))
  mx2v = jnp.broadcast_to(mx2.reshape(1), (16,))
  acc2, den2 = _sc_agg("edge", h2, h2, als2[:, 0], ald2[:, 0], mx2v, src, dst)

  h3, als3, ald3, mx3 = _tc_mid2(acc2[0], acc2[1],
                                 den2[0].reshape(-1, 1), den2[1].reshape(-1, 1),
                                 b2.reshape(1, -1),
                                 W3, a_src3.reshape(-1, 1), a_dst3.reshape(-1, 1))
  mx3v = jnp.broadcast_to(mx3.reshape(1), (16,))
  num3, den3 = _sc_final(h3[:, 0], als3[:, 0], ald3[:, 0], mx3v, src, dst)
  outp = _tc_fin(num3[0].reshape(-1, 1), num3[1].reshape(-1, 1),
                 den3[0].reshape(-1, 1), den3[1].reshape(-1, 1),
                 b3.reshape(1, 1))
  return outp[:N, 0]
